# Initial kernel scaffold; baseline (speedup 1.0000x reference)
import jax
import jax.numpy as jnp
from jax.experimental import pallas as pl


def kernel(x1, edge_index1, edge_attr1, batch1, x2, edge_index2, edge_attr2, batch2, data3, w1, b1, w2, b2, gat_W, gat_a_src, gat_a_dst, gat_b, gcn_W, gcn_b, fc1_W, fc1_b, fc2_W, fc2_b, out_W, out_b):
    raise NotImplementedError("write your pallas kernel here")


# speedup vs baseline: 1.4407x; 1.4407x over previous
import functools
import jax
import jax.numpy as jnp
from jax.experimental import pallas as pl
from jax.experimental.pallas import tpu as pltpu

NUM_GRAPHS = 256


def _dot(a, b):
    return jax.lax.dot_general(
        a, b, (((1,), (0,)), ((), ())),
        preferred_element_type=jnp.float32,
        precision=jax.lax.Precision.HIGHEST)


def _embed_body(x_ref, w1_ref, b1_ref, w2_ref, b2_ref, gw_ref, asrc_ref,
                adst_ref, h_ref, as_ref, ad_ref):
    x = x_ref[...]
    z = jnp.maximum(_dot(x, w1_ref[...]) + b1_ref[...][None, :], 0.0)
    z = jnp.maximum(_dot(z, w2_ref[...]) + b2_ref[...][None, :], 0.0)
    h = _dot(z, gw_ref[...])
    h_ref[...] = h
    as_ref[...] = _dot(h, asrc_ref[...])
    ad_ref[...] = _dot(h, adst_ref[...])


def _embed(x, w1, b1, w2, b2, gat_W, a_src, a_dst, bn=2000):
    n, f = x.shape
    return pl.pallas_call(
        _embed_body,
        grid=(n // bn,),
        in_specs=[
            pl.BlockSpec((bn, f), lambda i: (i, 0)),
            pl.BlockSpec(w1.shape, lambda i: (0, 0)),
            pl.BlockSpec(b1.shape, lambda i: (0,)),
            pl.BlockSpec(w2.shape, lambda i: (0, 0)),
            pl.BlockSpec(b2.shape, lambda i: (0,)),
            pl.BlockSpec(gat_W.shape, lambda i: (0, 0)),
            pl.BlockSpec((64, 1), lambda i: (0, 0)),
            pl.BlockSpec((64, 1), lambda i: (0, 0)),
        ],
        out_specs=[
            pl.BlockSpec((bn, 64), lambda i: (i, 0)),
            pl.BlockSpec((bn, 1), lambda i: (i, 0)),
            pl.BlockSpec((bn, 1), lambda i: (i, 0)),
        ],
        out_shape=[
            jax.ShapeDtypeStruct((n, 64), jnp.float32),
            jax.ShapeDtypeStruct((n, 1), jnp.float32),
            jax.ShapeDtypeStruct((n, 1), jnp.float32),
        ],
    )(x, w1, b1, w2, b2, gat_W, a_src[:, None], a_dst[:, None])


def _gatfin_body(num_ref, den_ref, h_ref, as_ref, ad_ref, gb_ref, gw_ref,
                 deg_ref, h2_ref, hh_ref, dinv_ref):
    e_self = as_ref[...] + ad_ref[...]
    e_self = jnp.maximum(e_self, 0.2 * e_self)
    ex_self = jnp.exp(e_self)
    num = num_ref[...] + ex_self * h_ref[...]
    den = den_ref[...] + ex_self
    gat = num / (den + 1e-16) + gb_ref[...][None, :]
    gat = jnp.maximum(gat, 0.0)
    h2 = _dot(gat, gw_ref[...])
    h2_ref[...] = h2
    dinv = jax.lax.rsqrt(deg_ref[...] + 1.0)
    dinv_ref[...] = dinv
    hh_ref[...] = h2 * dinv


def _gat_finish(num, den, h, as_, ad_, gat_b, gcn_W, deg, bn=2000):
    n = num.shape[0]
    return pl.pallas_call(
        _gatfin_body,
        grid=(n // bn,),
        in_specs=[
            pl.BlockSpec((bn, 64), lambda i: (i, 0)),
            pl.BlockSpec((bn, 1), lambda i: (i, 0)),
            pl.BlockSpec((bn, 64), lambda i: (i, 0)),
            pl.BlockSpec((bn, 1), lambda i: (i, 0)),
            pl.BlockSpec((bn, 1), lambda i: (i, 0)),
            pl.BlockSpec((64,), lambda i: (0,)),
            pl.BlockSpec((64, 128), lambda i: (0, 0)),
            pl.BlockSpec((bn, 1), lambda i: (i, 0)),
        ],
        out_specs=[
            pl.BlockSpec((bn, 128), lambda i: (i, 0)),
            pl.BlockSpec((bn, 128), lambda i: (i, 0)),
            pl.BlockSpec((bn, 1), lambda i: (i, 0)),
        ],
        out_shape=[
            jax.ShapeDtypeStruct((n, 128), jnp.float32),
            jax.ShapeDtypeStruct((n, 128), jnp.float32),
            jax.ShapeDtypeStruct((n, 1), jnp.float32),
        ],
    )(num, den, h, as_, ad_, gat_b, gcn_W, deg)


def _pool_body(acc_ref, hh_ref, dinv_ref, gb_ref, batch_ref, sum_ref,
               cnt_ref, mean_ref):
    i = pl.program_id(0)
    nblk = pl.num_programs(0)
    out2 = dinv_ref[...] * (acc_ref[...] + hh_ref[...]) + gb_ref[...][None, :]
    out2 = jnp.maximum(out2, 0.0)
    b = batch_ref[...]
    onehot = (b == jax.lax.broadcasted_iota(
        jnp.int32, (b.shape[0], NUM_GRAPHS), 1)).astype(jnp.float32)

    @pl.when(i == 0)
    def _():
        sum_ref[...] = jnp.zeros_like(sum_ref)
        cnt_ref[...] = jnp.zeros_like(cnt_ref)

    sum_ref[...] += jax.lax.dot_general(
        onehot, out2, (((0,), (0,)), ((), ())),
        preferred_element_type=jnp.float32,
        precision=jax.lax.Precision.HIGHEST)
    cnt_ref[...] += jnp.sum(onehot, axis=0)[:, None]

    @pl.when(i == nblk - 1)
    def _():
        mean_ref[...] = sum_ref[...] / jnp.maximum(cnt_ref[...], 1.0)


def _gcn_pool(acc, hh, dinv, gcn_b, batch, bn=2000):
    n = acc.shape[0]
    return pl.pallas_call(
        _pool_body,
        grid=(n // bn,),
        in_specs=[
            pl.BlockSpec((bn, 128), lambda i: (i, 0)),
            pl.BlockSpec((bn, 128), lambda i: (i, 0)),
            pl.BlockSpec((bn, 1), lambda i: (i, 0)),
            pl.BlockSpec((128,), lambda i: (0,)),
            pl.BlockSpec((bn, 1), lambda i: (i, 0)),
        ],
        out_specs=[
            pl.BlockSpec((NUM_GRAPHS, 128), lambda i: (0, 0)),
            pl.BlockSpec((NUM_GRAPHS, 1), lambda i: (0, 0)),
            pl.BlockSpec((NUM_GRAPHS, 128), lambda i: (0, 0)),
        ],
        out_shape=[
            jax.ShapeDtypeStruct((NUM_GRAPHS, 128), jnp.float32),
            jax.ShapeDtypeStruct((NUM_GRAPHS, 1), jnp.float32),
            jax.ShapeDtypeStruct((NUM_GRAPHS, 128), jnp.float32),
        ],
    )(acc, hh, dinv, gcn_b, batch[:, None])[2]


def _head_body(g_ref, w1_ref, b1_ref, w2_ref, b2_ref, w3_ref, b3_ref,
               out_ref, h1_ref):
    h1 = _dot(g_ref[...], w1_ref[...]) + b1_ref[...][None, :]
    h1_ref[...] = h1
    h = jnp.maximum(h1, 0.0)
    h = jnp.maximum(_dot(h, w2_ref[...]) + b2_ref[...][None, :], 0.0)
    out_ref[...] = _dot(h, w3_ref[...]) + b3_ref[...][None, :]


def _head(xcat, fc1_W, fc1_b, fc2_W, fc2_b, out_W, out_b):
    return pl.pallas_call(
        _head_body,
        out_shape=[
            jax.ShapeDtypeStruct((NUM_GRAPHS, 1), jnp.float32),
            jax.ShapeDtypeStruct((NUM_GRAPHS, 512), jnp.float32),
        ],
    )(xcat, fc1_W, fc1_b, fc2_W, fc2_b, out_W, out_b)


def _edge_passes(src, dst, h, as_, ad_, n):
    e = as_[src, 0] + ad_[dst, 0]
    e = jnp.maximum(e, 0.2 * e)
    ex = jnp.exp(e)
    den = jax.ops.segment_sum(ex, dst, num_segments=n)
    num = jax.ops.segment_sum(ex[:, None] * h[src], dst, num_segments=n)
    deg = jax.ops.segment_sum(jnp.ones_like(ex), dst, num_segments=n)
    return num, den[:, None], deg[:, None]


def _gcn_edge(src, dst, hh, n):
    return jax.ops.segment_sum(hh[src], dst, num_segments=n)


def _branch(x, ei, batch, w1, b1, w2, b2, gat_W, a_src, a_dst, gat_b,
            gcn_W, gcn_b):
    n = x.shape[0]
    h, as_, ad_ = _embed(x, w1, b1, w2, b2, gat_W, a_src, a_dst)
    src, dst = ei[0], ei[1]
    num, den, deg = _edge_passes(src, dst, h, as_, ad_, n)
    h2, hh, dinv = _gat_finish(num, den, h, as_, ad_, gat_b, gcn_W, deg)
    acc = _gcn_edge(src, dst, hh, n)
    return _gcn_pool(acc, hh, dinv, gcn_b, batch)


@jax.jit
def kernel(x1, edge_index1, edge_attr1, batch1, x2, edge_index2, edge_attr2,
           batch2, data3, w1, b1, w2, b2, gat_W, gat_a_src, gat_a_dst, gat_b,
           gcn_W, gcn_b, fc1_W, fc1_b, fc2_W, fc2_b, out_W, out_b):
    g1 = _branch(x1, edge_index1, batch1, w1, b1, w2, b2, gat_W, gat_a_src,
                 gat_a_dst, gat_b, gcn_W, gcn_b)
    g2 = _branch(x2, edge_index2, batch2, w1, b1, w2, b2, gat_W, gat_a_src,
                 gat_a_dst, gat_b, gcn_W, gcn_b)
    xcat = jnp.concatenate([g1, g2], axis=1)
    out, h1 = _head(xcat, fc1_W, fc1_b, fc2_W, fc2_b, out_W, out_b)
    return (out, h1)
